# Initial kernel scaffold; baseline (speedup 1.0000x reference)
import jax
import jax.numpy as jnp
from jax.experimental import pallas as pl


def kernel(x, edge_index, edge_attr, node_attr, batch, additional_message_features, Wq, Wk_fc1, Wk_fc2, Wv_fc1, Wv_fc2, Wdot):
    raise NotImplementedError("write your pallas kernel here")


# speedup vs baseline: 3.1785x; 3.1785x over previous
import functools

import jax
import jax.numpy as jnp
import numpy as np
from jax import lax
from jax.experimental import pallas as pl
from jax.experimental.pallas import tpu as pltpu
from jax.experimental.pallas import tpu_sc as plsc

N_NODES = 10000
N_EDGES = 160000
D = 16
NW = 32
ROW = 128
NR = 40
PER_W = NR * ROW
E_PAD = NW * PER_W
ROWS_PER_SUB = N_NODES // 16

EMB_SCALE = float(1.14136 * np.exp(2.0) * 4.0)
INV_STEP = 17.0 / 8.0

_rh = np.zeros((16, 256), np.float32)
_ri = np.zeros((16, 256), np.float32)
for _h in range(16):
    for _i in range(16):
        _rh[_h, _h * 16 + _i] = 1.0
        _ri[_i, _h * 16 + _i] = 1.0

_MESH = dict(core_axis_name="c", subcore_axis_name="s", num_cores=2, num_subcores=16)


def _gather_body(x_hbm, src_hbm, dst_hbm, xs_hbm, xd_hbm, idx_v, rows_v, sem):
    c = lax.axis_index("c")
    s = lax.axis_index("s")
    wid = s * 2 + c
    base = wid * PER_W
    for ind_hbm, out_hbm in ((src_hbm, xs_hbm), (dst_hbm, xd_hbm)):
        pltpu.sync_copy(ind_hbm.at[wid], idx_v)

        def grp(g, carry):
            handles = []
            for b in range(8):
                j = g * 8 + b
                handles.append(
                    pltpu.async_copy(
                        x_hbm.at[idx_v.at[j]], rows_v.at[pl.ds(j * ROW, ROW)], sem
                    )
                )
            for h in handles:
                h.wait()
            return carry

        lax.fori_loop(0, NR // 8, grp, 0)
        pltpu.sync_copy(rows_v, out_hbm.at[pl.ds(base, PER_W)])


def _make_gather():
    return functools.partial(
        pl.kernel,
        out_type=[
            jax.ShapeDtypeStruct((E_PAD, D), jnp.float32),
            jax.ShapeDtypeStruct((E_PAD, D), jnp.float32),
        ],
        mesh=plsc.VectorSubcoreMesh(**_MESH),
        scratch_types=[
            pltpu.VMEM((NR, ROW), jnp.int32),
            pltpu.VMEM((PER_W, D), jnp.float32),
            pltpu.SemaphoreType.DMA,
        ],
        compiler_params=pltpu.CompilerParams(use_tc_tiling_on_sc=False),
    )(_gather_body)


def _sus(t):
    safe = jnp.where(t > 0.0, t, 1.0)
    return jnp.where(t > 0.0, jnp.exp(-1.0 / safe), 0.0)


def _dense_body(el_ref, sh_ref, xs_ref, xd_ref, wk1_ref, wv1_ref, wq_ref,
                wdot_ref, ak_ref, av_ref, rh_ref, ri_ref, out_ref):
    el = el_ref[...]
    sh = sh_ref[...]
    xs = xs_ref[...]
    xd = xd_ref[...]
    j = lax.broadcasted_iota(jnp.int32, (1, 16), 1).astype(jnp.float32)
    diff = el * INV_STEP - (j + 1.0)
    emb = EMB_SCALE * _sus(diff + 1.0) * _sus(1.0 - diff)
    cutoff = _sus(10.0 - 1.25 * el)

    def mm(a, b):
        return jnp.dot(a, b, preferred_element_type=jnp.float32)

    def silu(t):
        return t / (1.0 + jnp.exp(-t))

    hk = silu(mm(emb, wk1_ref[...]) * 0.25)
    hv = silu(mm(emb, wv1_ref[...]) * 0.25)
    tile_xs = mm(xs, ri_ref[...])
    ok = mm(hk, rh_ref[...]) * tile_xs
    ov = mm(hv, rh_ref[...]) * tile_xs
    scale = sh * (1.0 / 16.0)
    k = mm(ok, ak_ref[...]) * scale
    v = mm(ov, av_ref[...]) * scale
    t = mm(mm(xd, wq_ref[...]), wdot_ref[...])
    dot = jnp.sum(t * k, axis=1, keepdims=True) * (1.0 / 64.0)
    w = cutoff * jnp.exp(dot)
    u = jnp.sqrt(w) * v
    out_ref[...] = jnp.concatenate([u, jnp.broadcast_to(w, u.shape)], axis=1)


def _dense_call(el, sh, xs, xd, wk1, wv1, wq, wdot, ak, av, rh, ri):
    B = 2048
    grid = (E_PAD // B,)
    edge = lambda i: (i, 0)
    full = lambda i: (0, 0)
    return pl.pallas_call(
        _dense_body,
        grid=grid,
        in_specs=[
            pl.BlockSpec((B, 1), edge),
            pl.BlockSpec((B, 1), edge),
            pl.BlockSpec((B, D), edge),
            pl.BlockSpec((B, D), edge),
            pl.BlockSpec((D, D), full),
            pl.BlockSpec((D, D), full),
            pl.BlockSpec((D, D), full),
            pl.BlockSpec((D, D), full),
            pl.BlockSpec((256, D), full),
            pl.BlockSpec((256, D), full),
            pl.BlockSpec((D, 256), full),
            pl.BlockSpec((D, 256), full),
        ],
        out_specs=pl.BlockSpec((B, 2 * D), edge),
        out_shape=jax.ShapeDtypeStruct((E_PAD, 2 * D), jnp.float32),
    )(el, sh, xs, xd, wk1, wv1, wq, wdot, ak, av, rh, ri)


def _scatter_body(dst_hbm, vals_hbm, zeros_hbm, part_hbm, table, idx_v, vals_v, sem):
    c = lax.axis_index("c")
    s = lax.axis_index("s")
    wid = s * 2 + c
    base = wid * PER_W
    pltpu.sync_copy(zeros_hbm, table.at[pl.ds(s * ROWS_PER_SUB, ROWS_PER_SUB)])
    plsc.subcore_barrier()
    pltpu.sync_copy(dst_hbm.at[wid], idx_v)

    def chunk(t, carry):
        pltpu.sync_copy(vals_hbm.at[pl.ds(base + t * (20 * ROW), 20 * ROW)], vals_v)
        for j in range(20):
            pltpu.sync_copy(
                vals_v.at[pl.ds(j * ROW, ROW)], table.at[idx_v.at[t * 20 + j]], add=True
            )
        return carry

    lax.fori_loop(0, 2, chunk, 0)
    plsc.subcore_barrier()
    pltpu.sync_copy(
        table.at[pl.ds(s * ROWS_PER_SUB, ROWS_PER_SUB)],
        part_hbm.at[c, pl.ds(s * ROWS_PER_SUB, ROWS_PER_SUB)],
    )


def _make_scatter():
    return functools.partial(
        pl.kernel,
        out_type=jax.ShapeDtypeStruct((2, N_NODES, 2 * D), jnp.float32),
        mesh=plsc.VectorSubcoreMesh(**_MESH),
        scratch_types=[
            pltpu.VMEM_SHARED((N_NODES, 2 * D), jnp.float32),
            pltpu.VMEM((NR, ROW), jnp.int32),
            pltpu.VMEM((20 * ROW, 2 * D), jnp.float32),
            pltpu.SemaphoreType.DMA,
        ],
        compiler_params=pltpu.CompilerParams(use_tc_tiling_on_sc=False),
    )(_scatter_body)


def _final_body(p_ref, out_ref):
    sacc = p_ref[0] + p_ref[1]
    u = sacc[:, :D]
    z = sacc[:, D:D + 1]
    zz = jnp.where(z == 0.0, 1.0, z)
    out_ref[...] = u / jnp.sqrt(zz)


def _final_call(parts):
    Bn = 2000
    return pl.pallas_call(
        _final_body,
        grid=(N_NODES // Bn,),
        in_specs=[pl.BlockSpec((2, Bn, 2 * D), lambda i: (0, i, 0))],
        out_specs=pl.BlockSpec((Bn, D), lambda i: (i, 0)),
        out_shape=jax.ShapeDtypeStruct((N_NODES, D), jnp.float32),
    )(parts)


def kernel(x, edge_index, edge_attr, node_attr, batch, additional_message_features,
           Wq, Wk_fc1, Wk_fc2, Wv_fc1, Wv_fc2, Wdot):
    del node_attr, batch
    pad = E_PAD - N_EDGES
    src = edge_index[0].astype(jnp.int32)
    dst = edge_index[1].astype(jnp.int32)
    src3 = jnp.concatenate([src, jnp.zeros((pad,), jnp.int32)]).reshape(NW, NR, ROW)
    dst3 = jnp.concatenate([dst, jnp.zeros((pad,), jnp.int32)]).reshape(NW, NR, ROW)
    el = jnp.concatenate(
        [additional_message_features[:, 0], jnp.full((pad,), 100.0, jnp.float32)]
    ).reshape(E_PAD, 1)
    sh = jnp.concatenate(
        [edge_attr[:, 0], jnp.zeros((pad,), jnp.float32)]
    ).reshape(E_PAD, 1)
    ak = Wk_fc2.reshape(256, D)
    av = Wv_fc2.reshape(256, D)
    rh = jnp.asarray(_rh)
    ri = jnp.asarray(_ri)
    zeros = jnp.zeros((ROWS_PER_SUB, 2 * D), jnp.float32)

    xs, xd = _make_gather()(x, src3, dst3)
    vals = _dense_call(el, sh, xs, xd, Wk_fc1, Wv_fc1, Wq, Wdot, ak, av, rh, ri)
    parts = _make_scatter()(dst3, vals, zeros)
    return _final_call(parts)


# speedup vs baseline: 5.1003x; 1.6046x over previous
import functools

import jax
import jax.numpy as jnp
import numpy as np
from jax import lax
from jax.experimental import pallas as pl
from jax.experimental.pallas import tpu as pltpu
from jax.experimental.pallas import tpu_sc as plsc

N_NODES = 10000
N_EDGES = 160000
D = 16
NW = 32
ROW = 128
NR = 40
PER_W = NR * ROW
E_PAD = NW * PER_W
ROWS_PER_SUB = N_NODES // 16

EMB_SCALE = float(1.14136 * np.exp(2.0) * 4.0)
INV_STEP = 17.0 / 8.0

_rh = np.zeros((16, 256), np.float32)
_ri = np.zeros((16, 256), np.float32)
for _h in range(16):
    for _i in range(16):
        _rh[_h, _h * 16 + _i] = 1.0
        _ri[_i, _h * 16 + _i] = 1.0

_MESH = dict(core_axis_name="c", subcore_axis_name="s", num_cores=2, num_subcores=16)


def _gather_body(x_hbm, src_hbm, dst_hbm, xs_hbm, xd_hbm, idx_v, rows_v, sem):
    c = lax.axis_index("c")
    s = lax.axis_index("s")
    wid = s * 2 + c
    base = wid * PER_W
    for ind_hbm, out_hbm in ((src_hbm, xs_hbm), (dst_hbm, xd_hbm)):
        pltpu.sync_copy(ind_hbm.at[wid], idx_v)

        def grp(g, carry):
            handles = []
            for b in range(8):
                j = g * 8 + b
                handles.append(
                    pltpu.async_copy(
                        x_hbm.at[idx_v.at[j]], rows_v.at[pl.ds(j * ROW, ROW)], sem
                    )
                )
            for h in handles:
                h.wait()
            return carry

        lax.fori_loop(0, NR // 8, grp, 0)
        pltpu.sync_copy(rows_v, out_hbm.at[pl.ds(base, PER_W)])


def _make_gather():
    return functools.partial(
        pl.kernel,
        out_type=[
            jax.ShapeDtypeStruct((E_PAD, D), jnp.float32),
            jax.ShapeDtypeStruct((E_PAD, D), jnp.float32),
        ],
        mesh=plsc.VectorSubcoreMesh(**_MESH),
        scratch_types=[
            pltpu.VMEM((NR, ROW), jnp.int32),
            pltpu.VMEM((PER_W, D), jnp.float32),
            pltpu.SemaphoreType.DMA,
        ],
        compiler_params=pltpu.CompilerParams(use_tc_tiling_on_sc=False),
    )(_gather_body)


def _sus(t):
    safe = jnp.where(t > 0.0, t, 1.0)
    return jnp.where(t > 0.0, jnp.exp(-1.0 / safe), 0.0)


def _mm_t(w, a):
    return lax.dot_general(w, a, (((0,), (0,)), ((), ())),
                           preferred_element_type=jnp.float32)


def _mm_r(w, a):
    return lax.dot_general(w, a, (((0,), (1,)), ((), ())),
                           preferred_element_type=jnp.float32)


def _dense_body(el_ref, sh_ref, xs_ref, xd_ref, wk1_ref, wv1_ref, wq_ref,
                wdot_ref, ak_ref, av_ref, rh_ref, ri_ref, out_ref):
    el = el_ref[...]
    sh = sh_ref[...]
    xs = xs_ref[...]
    xd = xd_ref[...]
    i16 = lax.broadcasted_iota(jnp.int32, (D, 1), 0).astype(jnp.float32)
    diff = el * INV_STEP - (i16 + 1.0)
    emb_t = EMB_SCALE * _sus(diff + 1.0) * _sus(1.0 - diff)
    cutoff = _sus(10.0 - 1.25 * el)

    def silu(t):
        return t / (1.0 + jnp.exp(-t))

    hk_t = silu(_mm_t(wk1_ref[...], emb_t) * 0.25)
    hv_t = silu(_mm_t(wv1_ref[...], emb_t) * 0.25)
    tile_xs_t = _mm_r(ri_ref[...], xs)
    ok_t = _mm_t(rh_ref[...], hk_t) * tile_xs_t
    ov_t = _mm_t(rh_ref[...], hv_t) * tile_xs_t
    scale = sh * (1.0 / 16.0)
    k_t = _mm_t(ak_ref[...], ok_t) * scale
    v_t = _mm_t(av_ref[...], ov_t) * scale
    t_t = _mm_t(wdot_ref[...], _mm_r(wq_ref[...], xd))
    dot = jnp.sum(t_t * k_t, axis=0, keepdims=True) * (1.0 / 64.0)
    w = cutoff * jnp.exp(dot)
    u_t = jnp.sqrt(w) * v_t
    out32_t = jnp.concatenate([u_t, jnp.broadcast_to(w, u_t.shape)], axis=0)
    r0 = lax.broadcasted_iota(jnp.int32, (2 * D, 2 * D), 0)
    r1 = lax.broadcasted_iota(jnp.int32, (2 * D, 2 * D), 1)
    eye = (r0 == r1).astype(jnp.float32)
    out_ref[...] = _mm_t(out32_t, eye)


def _dense_call(el, sh, xs, xd, wk1, wv1, wq, wdot, ak, av, rh, ri):
    B = 2048
    grid = (E_PAD // B,)
    edge = lambda i: (i, 0)
    lane = lambda i: (0, i)
    full = lambda i: (0, 0)
    return pl.pallas_call(
        _dense_body,
        grid=grid,
        in_specs=[
            pl.BlockSpec((1, B), lane),
            pl.BlockSpec((1, B), lane),
            pl.BlockSpec((B, D), edge),
            pl.BlockSpec((B, D), edge),
            pl.BlockSpec((D, D), full),
            pl.BlockSpec((D, D), full),
            pl.BlockSpec((D, D), full),
            pl.BlockSpec((D, D), full),
            pl.BlockSpec((256, D), full),
            pl.BlockSpec((256, D), full),
            pl.BlockSpec((D, 256), full),
            pl.BlockSpec((D, 256), full),
        ],
        out_specs=pl.BlockSpec((B, 2 * D), edge),
        out_shape=jax.ShapeDtypeStruct((E_PAD, 2 * D), jnp.float32),
    )(el, sh, xs, xd, wk1, wv1, wq, wdot, ak, av, rh, ri)


def _scatter_body(dst_hbm, vals_hbm, zeros_hbm, part_hbm, table, idx_v, vals_v, sem):
    c = lax.axis_index("c")
    s = lax.axis_index("s")
    wid = s * 2 + c
    base = wid * PER_W
    pltpu.sync_copy(zeros_hbm, table.at[pl.ds(s * ROWS_PER_SUB, ROWS_PER_SUB)])
    plsc.subcore_barrier()
    pltpu.sync_copy(dst_hbm.at[wid], idx_v)

    def chunk(t, carry):
        pltpu.sync_copy(vals_hbm.at[pl.ds(base + t * (20 * ROW), 20 * ROW)], vals_v)
        for j in range(20):
            pltpu.sync_copy(
                vals_v.at[pl.ds(j * ROW, ROW)], table.at[idx_v.at[t * 20 + j]], add=True
            )
        return carry

    lax.fori_loop(0, 2, chunk, 0)
    plsc.subcore_barrier()
    pltpu.sync_copy(
        table.at[pl.ds(s * ROWS_PER_SUB, ROWS_PER_SUB)],
        part_hbm.at[c, pl.ds(s * ROWS_PER_SUB, ROWS_PER_SUB)],
    )


def _make_scatter():
    return functools.partial(
        pl.kernel,
        out_type=jax.ShapeDtypeStruct((2, N_NODES, 2 * D), jnp.float32),
        mesh=plsc.VectorSubcoreMesh(**_MESH),
        scratch_types=[
            pltpu.VMEM_SHARED((N_NODES, 2 * D), jnp.float32),
            pltpu.VMEM((NR, ROW), jnp.int32),
            pltpu.VMEM((20 * ROW, 2 * D), jnp.float32),
            pltpu.SemaphoreType.DMA,
        ],
        compiler_params=pltpu.CompilerParams(use_tc_tiling_on_sc=False),
    )(_scatter_body)


def _final_body(p_ref, out_ref):
    sacc = p_ref[0] + p_ref[1]
    u = sacc[:, :D]
    z = sacc[:, D:D + 1]
    zz = jnp.where(z == 0.0, 1.0, z)
    out_ref[...] = u / jnp.sqrt(zz)


def _final_call(parts):
    Bn = 2000
    return pl.pallas_call(
        _final_body,
        grid=(N_NODES // Bn,),
        in_specs=[pl.BlockSpec((2, Bn, 2 * D), lambda i: (0, i, 0))],
        out_specs=pl.BlockSpec((Bn, D), lambda i: (i, 0)),
        out_shape=jax.ShapeDtypeStruct((N_NODES, D), jnp.float32),
    )(parts)


def kernel(x, edge_index, edge_attr, node_attr, batch, additional_message_features,
           Wq, Wk_fc1, Wk_fc2, Wv_fc1, Wv_fc2, Wdot):
    del node_attr, batch
    pad = E_PAD - N_EDGES
    src = edge_index[0].astype(jnp.int32)
    dst = edge_index[1].astype(jnp.int32)
    src3 = jnp.concatenate([src, jnp.zeros((pad,), jnp.int32)]).reshape(NW, NR, ROW)
    dst3 = jnp.concatenate([dst, jnp.zeros((pad,), jnp.int32)]).reshape(NW, NR, ROW)
    el = jnp.concatenate(
        [additional_message_features[:, 0], jnp.full((pad,), 100.0, jnp.float32)]
    ).reshape(1, E_PAD)
    sh = jnp.concatenate(
        [edge_attr[:, 0], jnp.zeros((pad,), jnp.float32)]
    ).reshape(1, E_PAD)
    ak = Wk_fc2.reshape(256, D)
    av = Wv_fc2.reshape(256, D)
    rh = jnp.asarray(_rh)
    ri = jnp.asarray(_ri)
    zeros = jnp.zeros((ROWS_PER_SUB, 2 * D), jnp.float32)

    xs, xd = _make_gather()(x, src3, dst3)
    vals = _dense_call(el, sh, xs, xd, Wk_fc1, Wv_fc1, Wq, Wdot, ak, av, rh, ri)
    parts = _make_scatter()(dst3, vals, zeros)
    return _final_call(parts)


# speedup vs baseline: 7.8517x; 1.5395x over previous
import functools

import jax
import jax.numpy as jnp
import numpy as np
from jax import lax
from jax.experimental import pallas as pl
from jax.experimental.pallas import tpu as pltpu
from jax.experimental.pallas import tpu_sc as plsc

N_NODES = 10000
N_EDGES = 160000
D = 16
NW = 32
ROW = 128
NR = 40
PER_W = NR * ROW
E_PAD = NW * PER_W
N_PAD = 10240
ROWS_PER_SUB = N_PAD // 16

EMB_SCALE = float(1.14136 * np.exp(2.0) * 4.0)
INV_STEP = 17.0 / 8.0

_rh = np.zeros((16, 256), np.float32)
_ri = np.zeros((16, 256), np.float32)
for _h in range(16):
    for _i in range(16):
        _rh[_h, _h * 16 + _i] = 1.0
        _ri[_i, _h * 16 + _i] = 1.0

_MESH = dict(core_axis_name="c", subcore_axis_name="s", num_cores=2, num_subcores=16)

DB = 8192
_q = np.arange(DB)
_ein = 8 * (_q % (DB // 8)) + _q // (DB // 8)
_t = np.arange(DB)
_eout = _ein[(DB // 4) * (_t % 4) + _t // 4]
_blk = np.arange(0, E_PAD, DB)[:, None]
_PERM_IN = (_blk + _ein[None, :]).reshape(-1).astype(np.int32)
_PERM_OUT = (_blk + _eout[None, :]).reshape(-1).astype(np.int32)


def _gather_body(nr, x_hbm, src_hbm, dst_hbm, xs_hbm, xd_hbm, idx_v, rows_v, sem):
    per_w = nr * ROW
    c = lax.axis_index("c")
    s = lax.axis_index("s")
    wid = s * 2 + c
    base = wid * per_w
    for ind_hbm, out_hbm in ((src_hbm, xs_hbm), (dst_hbm, xd_hbm)):
        pltpu.sync_copy(ind_hbm.at[wid], idx_v)

        def grp(g, carry):
            handles = []
            for b in range(4):
                j = g * 4 + b
                handles.append(
                    pltpu.async_copy(
                        x_hbm.at[idx_v.at[j]], rows_v.at[pl.ds(j * ROW, ROW)], sem
                    )
                )
            for h in handles:
                h.wait()
            return carry

        lax.fori_loop(0, nr // 4, grp, 0)
        pltpu.sync_copy(rows_v, out_hbm.at[pl.ds(base, per_w)])


def _make_gather(e_half):
    nr = e_half // (NW * ROW)
    return functools.partial(
        pl.kernel,
        out_type=[
            jax.ShapeDtypeStruct((e_half, D), jnp.float32),
            jax.ShapeDtypeStruct((e_half, D), jnp.float32),
        ],
        mesh=plsc.VectorSubcoreMesh(**_MESH),
        scratch_types=[
            pltpu.VMEM((nr, ROW), jnp.int32),
            pltpu.VMEM((nr * ROW, D), jnp.float32),
            pltpu.SemaphoreType.DMA,
        ],
        compiler_params=pltpu.CompilerParams(use_tc_tiling_on_sc=False),
    )(functools.partial(_gather_body, nr))


def _sus(t):
    safe = jnp.where(t > 0.0, t, 1.0)
    return jnp.where(t > 0.0, jnp.exp(-1.0 / safe), 0.0)


def _mm_t(w, a):
    return lax.dot_general(w, a, (((0,), (0,)), ((), ())),
                           preferred_element_type=jnp.float32)


def _mm_r(w, a):
    return lax.dot_general(w, a, (((0,), (1,)), ((), ())),
                           preferred_element_type=jnp.float32)


def _dense_body(el_ref, sh_ref, xs_ref, xd_ref, wk1_ref, wv1_ref, wq_ref,
                wdot_ref, ak_ref, av_ref, rh_ref, ri_ref, out_ref):
    B = xs_ref.shape[0] * 128 // D
    el = el_ref[...]
    sh = sh_ref[...]
    xsp = xs_ref[...]
    xdp = xd_ref[...]
    xs = jnp.concatenate([xsp[:, D * p:D * (p + 1)] for p in range(8)], axis=0)
    xd = jnp.concatenate([xdp[:, D * p:D * (p + 1)] for p in range(8)], axis=0)
    i16 = lax.broadcasted_iota(jnp.int32, (D, 1), 0).astype(jnp.float32)
    diff = el * INV_STEP - (i16 + 1.0)
    emb_t = EMB_SCALE * _sus(diff + 1.0) * _sus(1.0 - diff)
    cutoff = _sus(10.0 - 1.25 * el)

    def silu(t):
        return t / (1.0 + jnp.exp(-t))

    hk_t = silu(_mm_t(wk1_ref[...], emb_t) * 0.25)
    hv_t = silu(_mm_t(wv1_ref[...], emb_t) * 0.25)
    tile_xs_t = _mm_r(ri_ref[...], xs)
    ok_t = _mm_t(rh_ref[...], hk_t) * tile_xs_t
    ov_t = _mm_t(rh_ref[...], hv_t) * tile_xs_t
    scale = sh * (1.0 / 16.0)
    k_t = _mm_t(ak_ref[...], ok_t) * scale
    v_t = _mm_t(av_ref[...], ov_t) * scale
    t_t = _mm_t(wdot_ref[...], _mm_r(wq_ref[...], xd))
    dot = jnp.sum(t_t * k_t, axis=0, keepdims=True) * (1.0 / 64.0)
    w = cutoff * jnp.exp(dot)
    u_t = jnp.sqrt(w) * v_t
    out32_t = jnp.concatenate([u_t, jnp.broadcast_to(w, u_t.shape)], axis=0)
    r0 = lax.broadcasted_iota(jnp.int32, (2 * D, 2 * D), 0)
    r1 = lax.broadcasted_iota(jnp.int32, (2 * D, 2 * D), 1)
    eye = (r0 == r1).astype(jnp.float32)
    out32 = _mm_t(out32_t, eye)
    q = B // 4
    out_ref[...] = jnp.concatenate([out32[q * a:q * (a + 1), :] for a in range(4)],
                                   axis=1)


def _dense_call(el, sh, xs, xd, wk1, wv1, wq, wdot, ak, av, rh, ri):
    B = DB
    e_half = el.shape[1]
    grid = (e_half // B,)
    edge = lambda i: (i, 0)
    lane = lambda i: (0, i)
    full = lambda i: (0, 0)
    return pl.pallas_call(
        _dense_body,
        grid=grid,
        in_specs=[
            pl.BlockSpec((1, B), lane),
            pl.BlockSpec((1, B), lane),
            pl.BlockSpec((B * D // 128, 128), edge),
            pl.BlockSpec((B * D // 128, 128), edge),
            pl.BlockSpec((D, D), full),
            pl.BlockSpec((D, D), full),
            pl.BlockSpec((D, D), full),
            pl.BlockSpec((D, D), full),
            pl.BlockSpec((256, D), full),
            pl.BlockSpec((256, D), full),
            pl.BlockSpec((D, 256), full),
            pl.BlockSpec((D, 256), full),
        ],
        out_specs=pl.BlockSpec((B * 2 * D // 128, 128), edge),
        out_shape=jax.ShapeDtypeStruct((e_half * 2 * D // 128, 128), jnp.float32),
    )(el, sh, xs, xd, wk1, wv1, wq, wdot, ak, av, rh, ri)


def _scatter_body(dst_hbm, vals_hbm, init_hbm, part_hbm, table, idx_v, vals_v, sem):
    c = lax.axis_index("c")
    s = lax.axis_index("s")
    wid = s * 2 + c
    pltpu.sync_copy(
        init_hbm.at[c, pl.ds(s * ROWS_PER_SUB, ROWS_PER_SUB)],
        table.at[pl.ds(s * ROWS_PER_SUB, ROWS_PER_SUB)],
    )
    plsc.subcore_barrier()
    pltpu.sync_copy(dst_hbm.at[wid], idx_v)
    pltpu.sync_copy(vals_hbm.at[pl.ds(wid * (20 * ROW), 20 * ROW)], vals_v)
    for j in range(20):
        pltpu.sync_copy(
            vals_v.at[pl.ds(j * ROW, ROW)], table.at[idx_v.at[j]], add=True
        )
    plsc.subcore_barrier()
    pltpu.sync_copy(
        table.at[pl.ds(s * ROWS_PER_SUB, ROWS_PER_SUB)],
        part_hbm.at[c, pl.ds(s * ROWS_PER_SUB, ROWS_PER_SUB)],
    )


def _make_scatter():
    return functools.partial(
        pl.kernel,
        out_type=jax.ShapeDtypeStruct((2, N_PAD, 2 * D), jnp.float32),
        mesh=plsc.VectorSubcoreMesh(**_MESH),
        scratch_types=[
            pltpu.VMEM_SHARED((N_PAD, 2 * D), jnp.float32),
            pltpu.VMEM((NR // 2, ROW), jnp.int32),
            pltpu.VMEM((20 * ROW, 2 * D), jnp.float32),
            pltpu.SemaphoreType.DMA,
        ],
        compiler_params=pltpu.CompilerParams(use_tc_tiling_on_sc=False),
    )(_scatter_body)


def _final_body(p_ref, out_ref):
    sacc = p_ref[0] + p_ref[1]
    outs = []
    for a in range(4):
        piece = sacc[:, 32 * a:32 * (a + 1)]
        u = piece[:, :D]
        z = piece[:, D:D + 1]
        zz = jnp.where(z == 0.0, 1.0, z)
        outs.append(u / jnp.sqrt(zz))
    out_ref[...] = jnp.concatenate(outs, axis=1)


def _final_call(parts):
    rows = 512
    nrows = N_PAD * 2 * D // 128
    return pl.pallas_call(
        _final_body,
        grid=(nrows // rows,),
        in_specs=[pl.BlockSpec((2, rows, 128), lambda i: (0, i, 0))],
        out_specs=pl.BlockSpec((rows, 4 * D), lambda i: (i, 0)),
        out_shape=jax.ShapeDtypeStruct((nrows, 4 * D), jnp.float32),
    )(parts)


def kernel(x, edge_index, edge_attr, node_attr, batch, additional_message_features,
           Wq, Wk_fc1, Wk_fc2, Wv_fc1, Wv_fc2, Wdot):
    del node_attr, batch
    pad = E_PAD - N_EDGES
    src = edge_index[0].astype(jnp.int32)
    dst = edge_index[1].astype(jnp.int32)
    src3 = jnp.concatenate([src, jnp.zeros((pad,), jnp.int32)]).reshape(NW, NR, ROW)
    dst_p = jnp.concatenate([dst, jnp.zeros((pad,), jnp.int32)])
    dst3 = dst_p.reshape(NW, NR, ROW)
    dst3_sc = dst_p[jnp.asarray(_PERM_OUT)].reshape(NW, NR, ROW)
    el = jnp.concatenate(
        [additional_message_features[:, 0], jnp.full((pad,), 100.0, jnp.float32)]
    )[jnp.asarray(_PERM_IN)].reshape(1, E_PAD)
    sh = jnp.concatenate(
        [edge_attr[:, 0], jnp.zeros((pad,), jnp.float32)]
    )[jnp.asarray(_PERM_IN)].reshape(1, E_PAD)
    ak = Wk_fc2.reshape(256, D)
    av = Wv_fc2.reshape(256, D)
    rh = jnp.asarray(_rh)
    ri = jnp.asarray(_ri)
    zeros = jnp.zeros((2, N_PAD, 2 * D), jnp.float32)

    eh = E_PAD // 2
    nrh = NR // 2
    gather = _make_gather(eh)
    scatter = _make_scatter()
    dst_sc_flat = dst3_sc.reshape(2, eh)
    parts = zeros
    for h in range(2):
        s3 = src3.reshape(2, eh)[h].reshape(NW, nrh, ROW)
        d3 = dst3.reshape(2, eh)[h].reshape(NW, nrh, ROW)
        xs, xd = gather(x, s3, d3)
        vals = _dense_call(
            el[:, h * eh:(h + 1) * eh], sh[:, h * eh:(h + 1) * eh],
            xs.reshape(eh * D // 128, 128), xd.reshape(eh * D // 128, 128),
            Wk_fc1, Wv_fc1, Wq, Wdot, ak, av, rh, ri,
        )
        parts = scatter(
            dst_sc_flat[h].reshape(NW, nrh, ROW), vals.reshape(eh, 2 * D), parts
        )
    parts128 = parts.reshape(2, N_PAD * 2 * D // 128, 128)
    return _final_call(parts128).reshape(N_PAD, D)[:N_NODES]
